# Initial kernel scaffold; baseline (speedup 1.0000x reference)
import jax
import jax.numpy as jnp
from jax.experimental import pallas as pl


def kernel(feature, edge_index, num_nodes, W, b):
    raise NotImplementedError("write your pallas kernel here")


# speedup vs baseline: 21.9646x; 21.9646x over previous
import functools

import jax
import jax.numpy as jnp
from jax import lax
from jax.experimental import pallas as pl
from jax.experimental.pallas import tpu as pltpu
from jax.experimental.pallas import tpu_sc as plsc

N = 10000
D = 256
E = 160000
B = 100
NPG = 100
K = 50

NC = 2
NS = 16
L = 16
NW = NC * NS

NP = 10240
NROWS = NP // L
EPW = E // NW
FULL_ITERS = EPW // L

GROWS = 5120 // L
BPW = 5120 // NW


def _iota16():
    return lax.broadcasted_iota(jnp.int32, (L,), 0)


def _zero_flat(acc_ref, n):
    z = jnp.zeros((L,), jnp.float32)

    def body(i, _):
        acc_ref[pl.ds(i * L, L)] = z
        return 0

    lax.fori_loop(0, n // L, body, 0)


def _tree_reduce(sid, cid, acc_ref, shared_ref, tmp_ref, out_hbm, n):
    chunk = n // NS
    pltpu.sync_copy(acc_ref, shared_ref.at[sid])
    plsc.subcore_barrier()
    lo = sid * chunk
    pltpu.sync_copy(shared_ref.at[0, pl.ds(lo, chunk)], acc_ref.at[pl.ds(0, chunk)])
    for t in range(1, NS):
        pltpu.sync_copy(shared_ref.at[t, pl.ds(lo, chunk)], tmp_ref)

        def body(i, _):
            acc_ref[pl.ds(i * L, L)] = acc_ref[pl.ds(i * L, L)] + tmp_ref[pl.ds(i * L, L)]
            return 0

        lax.fori_loop(0, chunk // L, body, 0)
    pltpu.sync_copy(acc_ref.at[pl.ds(0, chunk)], out_hbm.at[cid, pl.ds(lo, chunk)])


@functools.partial(
    pl.kernel,
    out_type=jax.ShapeDtypeStruct((NC, 2 * NP), jnp.float32),
    mesh=plsc.VectorSubcoreMesh(core_axis_name="c", subcore_axis_name="s",
                                num_cores=NC, num_subcores=NS),
    compiler_params=pltpu.CompilerParams(needs_layout_passes=False),
    scratch_types=[
        pltpu.VMEM((EPW,), jnp.int32),
        pltpu.VMEM((EPW,), jnp.int32),
        pltpu.VMEM((2 * NP,), jnp.float32),
        pltpu.VMEM((2 * NP // NS,), jnp.float32),
        pltpu.VMEM_SHARED((NS, 2 * NP), jnp.float32),
    ],
)
def _sc_degrees(src_hbm, dst_hbm, out_hbm, src_v, dst_v, acc_v, tmp_v,
                shared_v):
    cid = lax.axis_index("c")
    sid = lax.axis_index("s")
    wid = sid * NC + cid
    base = wid * EPW

    pltpu.sync_copy(src_hbm.at[pl.ds(base, EPW)], src_v)
    pltpu.sync_copy(dst_hbm.at[pl.ds(base, EPW)], dst_v)

    _zero_flat(acc_v, 2 * NP)

    ones = jnp.ones((L,), jnp.float32)

    def step(off, mask):
        s16 = src_v[pl.ds(off, L)]
        d16 = dst_v[pl.ds(off, L)]
        plsc.addupdate_scatter(acc_v, [s16], ones, mask=mask)
        plsc.addupdate_scatter(acc_v, [d16 + NP], ones, mask=mask)

    def body(i, _):
        step(i * L, None)
        return 0

    lax.fori_loop(0, FULL_ITERS, body, 0)
    step(EPW - L, _iota16() >= L - (EPW - FULL_ITERS * L))

    _tree_reduce(sid, cid, acc_v, shared_v, tmp_v, out_hbm, 2 * NP)


@functools.partial(
    pl.kernel,
    out_type=jax.ShapeDtypeStruct((NC, NP), jnp.float32),
    mesh=plsc.VectorSubcoreMesh(core_axis_name="c", subcore_axis_name="s",
                                num_cores=NC, num_subcores=NS),
    compiler_params=pltpu.CompilerParams(needs_layout_passes=False),
    scratch_types=[
        pltpu.VMEM((EPW,), jnp.int32),
        pltpu.VMEM((EPW,), jnp.int32),
        pltpu.VMEM((N,), jnp.float32),
        pltpu.VMEM((NP,), jnp.float32),
        pltpu.VMEM((NP // NS,), jnp.float32),
        pltpu.VMEM_SHARED((NS, NP), jnp.float32),
    ],
)
def _sc_aggregate(src_hbm, dst_hbm, y_hbm, out_hbm, src_v, dst_v, y_v, acc_v,
                  tmp_v, shared_v):
    cid = lax.axis_index("c")
    sid = lax.axis_index("s")
    wid = sid * NC + cid
    base = wid * EPW

    pltpu.sync_copy(src_hbm.at[pl.ds(base, EPW)], src_v)
    pltpu.sync_copy(dst_hbm.at[pl.ds(base, EPW)], dst_v)
    pltpu.sync_copy(y_hbm, y_v)

    _zero_flat(acc_v, NP)

    def step(off, mask):
        s16 = src_v[pl.ds(off, L)]
        d16 = dst_v[pl.ds(off, L)]
        vals = plsc.load_gather(y_v, [s16], mask=mask)
        plsc.addupdate_scatter(acc_v, [d16], vals, mask=mask)

    def body(i, _):
        step(i * L, None)
        return 0

    lax.fori_loop(0, FULL_ITERS, body, 0)
    step(EPW - L, _iota16() >= L - (EPW - FULL_ITERS * L))

    _tree_reduce(sid, cid, acc_v, shared_v, tmp_v, out_hbm, NP)


@functools.partial(
    pl.kernel,
    out_type=jax.ShapeDtypeStruct((NW * BPW, D), jnp.float32),
    mesh=plsc.VectorSubcoreMesh(core_axis_name="c", subcore_axis_name="s",
                                num_cores=NC, num_subcores=NS),
    compiler_params=pltpu.CompilerParams(needs_layout_passes=False),
    scratch_types=[
        pltpu.VMEM((2, 80), jnp.int32),
        pltpu.VMEM((BPW,), jnp.float32),
        pltpu.VMEM((BPW, D), jnp.float32),
        pltpu.SemaphoreType.DMA,
    ],
)
def _sc_gather_scale(table_hbm, idx_hbm, scale_hbm, out_hbm, idx_v, scale_v,
                     rows_v, sem):
    cid = lax.axis_index("c")
    sid = lax.axis_index("s")
    wid = sid * NC + cid
    base = wid * BPW

    pltpu.sync_copy(idx_hbm.at[pl.ds(wid * 2, 2)], idx_v)
    pltpu.sync_copy(scale_hbm.at[pl.ds(base, BPW)], scale_v)

    cp0 = pltpu.async_copy(table_hbm.at[idx_v.at[0]],
                           rows_v.at[pl.ds(0, 80)], sem)
    cp1 = pltpu.async_copy(table_hbm.at[idx_v.at[1]],
                           rows_v.at[pl.ds(80, 80)], sem)
    cp0.wait()
    cp1.wait()

    def body(i, _):
        s = plsc.load_gather(scale_v, [jnp.full((L,), i, jnp.int32)])
        for c in range(D // L):
            rows_v[i, pl.ds(c * L, L)] = rows_v[i, pl.ds(c * L, L)] * s
        return 0

    lax.fori_loop(0, BPW, body, 0)

    pltpu.sync_copy(rows_v, out_hbm.at[pl.ds(base, BPW)])


def _tc_y_body(x_ref, w_ref, deg_ref, y_ref):
    xw = jnp.dot(x_ref[...], w_ref[...], preferred_element_type=jnp.float32)
    d = deg_ref[0] + deg_ref[1]
    y_ref[...] = xw * lax.rsqrt(jnp.maximum(d, 1.0))


def _tc_y(feature, W, deg_src):
    blk = 1000
    return pl.pallas_call(
        _tc_y_body,
        grid=(N // blk,),
        in_specs=[
            pl.BlockSpec((blk, D), lambda i: (i, 0)),
            pl.BlockSpec((D, 1), lambda i: (0, 0)),
            pl.BlockSpec((2, blk, 1), lambda i: (0, i, 0)),
        ],
        out_specs=pl.BlockSpec((blk, 1), lambda i: (i, 0)),
        out_shape=jax.ShapeDtypeStruct((N, 1), jnp.float32),
    )(feature, W, deg_src)


def _tc_rank_body(agg_ref, din_ref, b_ref, perm_ref, scale_ref):
    s = (agg_ref[0] + agg_ref[1]) * lax.rsqrt(
        jnp.maximum(din_ref[0] + din_ref[1], 1.0)) + b_ref[0, 0]
    i_iota = lax.broadcasted_iota(jnp.int32, (B, NPG, NPG), 1)
    j_iota = lax.broadcasted_iota(jnp.int32, (B, NPG, NPG), 2)
    s_i = s[:, :, None]
    s_j = s[:, None, :]
    before = (s_j > s_i) | ((s_j == s_i) & (j_iota < i_iota))
    rank = jnp.sum(before.astype(jnp.int32), axis=2)
    r_iota = lax.broadcasted_iota(jnp.int32, (B, NPG, K), 2)
    m = rank[:, :, None] == r_iota
    col = lax.broadcasted_iota(jnp.int32, (B, NPG, K), 1)
    row = lax.broadcasted_iota(jnp.int32, (B, K), 0)
    perm_ref[...] = jnp.sum(jnp.where(m, col, 0), axis=1) + row * NPG
    ssel = jnp.sum(jnp.where(m, s[:, :, None], 0.0), axis=1)
    scale_ref[...] = jnp.tanh(ssel)


def _tc_rank(agg3, din3, b2):
    return pl.pallas_call(
        _tc_rank_body,
        out_shape=(jax.ShapeDtypeStruct((B, K), jnp.int32),
                   jax.ShapeDtypeStruct((B, K), jnp.float32)),
    )(agg3, din3, b2)


def kernel(feature, edge_index, num_nodes, W, b):
    src = edge_index[0]
    dst = edge_index[1]

    deg2 = _sc_degrees(src, dst)
    deg_src = deg2[:, :N].reshape(NC, N, 1)
    din3 = deg2[:, NP:NP + N].reshape(NC, B, NPG)

    y = _tc_y(feature, W, deg_src).reshape(N)

    aggp = _sc_aggregate(src, dst, y)
    agg3 = aggp[:, :N].reshape(NC, B, NPG)

    perm2, scale2 = _tc_rank(agg3, din3, b.reshape(1, 1))
    permf = perm2.reshape(-1)

    pad = NW * BPW - B * K
    idx_pad = jnp.concatenate(
        [permf, jnp.zeros((pad,), jnp.int32)]).reshape(NW * 2, 80)
    scale_pad = jnp.concatenate(
        [scale2.reshape(-1), jnp.zeros((pad,), jnp.float32)])

    gathered = _sc_gather_scale(feature, idx_pad, scale_pad)
    feat_out = gathered[:B * K]

    k = jnp.ceil(0.5 * num_nodes.astype(jnp.float32)).astype(jnp.int32)
    return feat_out, permf, k
